# baseline (device time: 166940 ns/iter reference)
import jax
import jax.numpy as jnp
from jax import lax
from jax.experimental import pallas as pl
from jax.experimental.pallas import tpu as pltpu

N_PAGES_GLOBAL = 128
N_PAGES_SHARD = 64
BS = 16
B = 8
H = 8
D = 64
NB = 64
SCALE = D ** -0.5


def kernel(Q, K, V, bt, lens):
    def body(q_ref, k_ref, v_ref, bt_ref, lens_ref, out_ref,
             k_full, v_full, vg, s_ref, send_sems, recv_sems):
        mx = lax.axis_index("x")
        my = lax.axis_index("y")
        mz = lax.axis_index("z")
        nbr = (1 - mx, my, mz)

        barrier_sem = pltpu.get_barrier_semaphore()
        pl.semaphore_signal(barrier_sem, inc=1, device_id=nbr,
                            device_id_type=pl.DeviceIdType.MESH)
        pl.semaphore_wait(barrier_sem, 1)

        my_off = mx * N_PAGES_SHARD
        k_rdma = pltpu.make_async_remote_copy(
            src_ref=k_ref,
            dst_ref=k_full.at[pl.ds(my_off, N_PAGES_SHARD)],
            send_sem=send_sems.at[0],
            recv_sem=recv_sems.at[0],
            device_id=nbr,
            device_id_type=pl.DeviceIdType.MESH,
        )
        v_rdma = pltpu.make_async_remote_copy(
            src_ref=v_ref,
            dst_ref=v_full.at[pl.ds(my_off, N_PAGES_SHARD)],
            send_sem=send_sems.at[1],
            recv_sem=recv_sems.at[1],
            device_id=nbr,
            device_id_type=pl.DeviceIdType.MESH,
        )
        k_rdma.start()
        v_rdma.start()

        k_full[pl.ds(my_off, N_PAGES_SHARD)] = k_ref[...]
        v_full[pl.ds(my_off, N_PAGES_SHARD)] = v_ref[...]

        k_rdma.wait()
        v_rdma.wait()

        for i in range(B):
            q = q_ref[i, 0]

            def gather_body(j, _, i=i, q=q):
                page = bt_ref[i, j]
                kb = k_full[page]
                sb = jnp.sum(kb * q[None, :, :], axis=-1) * SCALE
                s_ref[pl.ds(j * BS, BS)] = sb
                vg[pl.ds(j * BS, BS)] = v_full[page]
                return 0

            lax.fori_loop(0, NB, gather_body, 0)

            n_valid = lens_ref[i] * BS
            pos = lax.broadcasted_iota(jnp.int32, (NB * BS, H), 0)
            s = jnp.where(pos < n_valid, s_ref[...], -1e30)
            m = jnp.max(s, axis=0)
            p = jnp.exp(s - m[None, :])
            denom = jnp.sum(p, axis=0)
            p = p / denom[None, :]
            acc = jnp.sum(vg[...] * p[:, :, None], axis=0)
            out_ref[i, 0] = acc

    out_shape = jax.ShapeDtypeStruct((B, 1, H, D), jnp.float32)
    return pl.pallas_call(
        body,
        out_shape=out_shape,
        in_specs=[
            pl.BlockSpec(memory_space=pltpu.VMEM),
            pl.BlockSpec(memory_space=pltpu.VMEM),
            pl.BlockSpec(memory_space=pltpu.VMEM),
            pl.BlockSpec(memory_space=pltpu.SMEM),
            pl.BlockSpec(memory_space=pltpu.SMEM),
        ],
        out_specs=pl.BlockSpec(memory_space=pltpu.VMEM),
        scratch_shapes=[
            pltpu.VMEM((N_PAGES_GLOBAL, BS, H, D), jnp.float32),
            pltpu.VMEM((N_PAGES_GLOBAL, BS, H, D), jnp.float32),
            pltpu.VMEM((NB * BS, H, D), jnp.float32),
            pltpu.VMEM((NB * BS, H), jnp.float32),
            pltpu.SemaphoreType.DMA((2,)),
            pltpu.SemaphoreType.DMA((2,)),
        ],
        compiler_params=pltpu.CompilerParams(collective_id=0),
    )(Q, K, V, bt, lens)


# device time: 39413 ns/iter; 4.2357x vs baseline; 4.2357x over previous
import jax
import jax.numpy as jnp
from jax import lax
from jax.experimental import pallas as pl
from jax.experimental.pallas import tpu as pltpu

N_PAGES_SHARD = 64
BS = 16
B = 8
H = 8
D = 64
NB = 64
SCALE = D ** -0.5
NEG = -1e30


def kernel(Q, K, V, bt, lens):
    def body(q_ref, k_ref, v_ref, bt_ref, lens_ref, bt_v_ref, out_ref,
             psend, precv, send_sem, recv_sem):
        mx = lax.axis_index("x")
        my = lax.axis_index("y")
        mz = lax.axis_index("z")
        nbr = (1 - mx, my, mz)

        barrier_sem = pltpu.get_barrier_semaphore()
        pl.semaphore_signal(barrier_sem, inc=1, device_id=nbr,
                            device_id_type=pl.DeviceIdType.MESH)
        pl.semaphore_wait(barrier_sem, 1)

        kv = k_ref[...]
        vv = v_ref[...]

        for i in range(B):
            q = q_ref[i, 0]

            page_iota = lax.broadcasted_iota(jnp.int32, (N_PAGES_SHARD, NB), 0)
            slot_iota = lax.broadcasted_iota(jnp.int32, (N_PAGES_SHARD, NB), 1)
            bt_row = bt_v_ref[pl.ds(i, 1), :]
            match = (bt_row == page_iota + mx * N_PAGES_SHARD) & (
                slot_iota < lens_ref[i]
            )
            c = jnp.sum(match.astype(jnp.float32), axis=1)
            c3 = c[:, None, None]

            s = jnp.sum(kv * q[None, None, :, :], axis=-1) * SCALE
            s_m = jnp.where(c3 > 0.0, s, NEG)

            m_loc = jnp.max(jnp.max(s_m, axis=0), axis=0)
            e = jnp.exp(s_m - m_loc[None, None, :])
            w = c3 * e
            l_loc = jnp.sum(jnp.sum(w, axis=0), axis=0)
            acc = jnp.sum(jnp.sum(w[..., None] * vv, axis=0), axis=0)

            psend[i, 0] = acc
            psend[i, 1] = jnp.broadcast_to(m_loc[:, None], (H, D))
            psend[i, 2] = jnp.broadcast_to(l_loc[:, None], (H, D))

        rdma = pltpu.make_async_remote_copy(
            src_ref=psend,
            dst_ref=precv,
            send_sem=send_sem,
            recv_sem=recv_sem,
            device_id=nbr,
            device_id_type=pl.DeviceIdType.MESH,
        )
        rdma.start()
        rdma.wait()

        for i in range(B):
            acc0, m0, l0 = psend[i, 0], psend[i, 1], psend[i, 2]
            acc1, m1, l1 = precv[i, 0], precv[i, 1], precv[i, 2]
            m_new = jnp.maximum(m0, m1)
            e0 = jnp.exp(m0 - m_new)
            e1 = jnp.exp(m1 - m_new)
            denom = l0 * e0 + l1 * e1
            out_ref[i, 0] = (acc0 * e0 + acc1 * e1) / denom

    out_shape = jax.ShapeDtypeStruct((B, 1, H, D), jnp.float32)
    return pl.pallas_call(
        body,
        out_shape=out_shape,
        in_specs=[
            pl.BlockSpec(memory_space=pltpu.VMEM),
            pl.BlockSpec(memory_space=pltpu.VMEM),
            pl.BlockSpec(memory_space=pltpu.VMEM),
            pl.BlockSpec(memory_space=pltpu.SMEM),
            pl.BlockSpec(memory_space=pltpu.SMEM),
            pl.BlockSpec(memory_space=pltpu.VMEM),
        ],
        out_specs=pl.BlockSpec(memory_space=pltpu.VMEM),
        scratch_shapes=[
            pltpu.VMEM((B, 3, H, D), jnp.float32),
            pltpu.VMEM((B, 3, H, D), jnp.float32),
            pltpu.SemaphoreType.DMA,
            pltpu.SemaphoreType.DMA,
        ],
        compiler_params=pltpu.CompilerParams(collective_id=0),
    )(Q, K, V, bt, lens, bt)


# device time: 23404 ns/iter; 7.1330x vs baseline; 1.6840x over previous
import jax
import jax.numpy as jnp
from jax import lax
from jax.experimental import pallas as pl
from jax.experimental.pallas import tpu as pltpu

N_PAGES_SHARD = 64
BS = 16
B = 8
BL = 2
H = 8
D = 64
NB = 64
SCALE = D ** -0.5
NEG = -1e30


def kernel(Q, K, V, bt, lens):
    def body(q_ref, k_ref, v_ref, bt_ref, lens_ref, bt_v_ref, out_ref,
             psend, precv, send_sems, recv_sems):
        mx = lax.axis_index("x")
        my = lax.axis_index("y")
        mz = lax.axis_index("z")
        x_nbr = (1 - mx, my, mz)
        y_nbr = (mx, 1 - my, mz)
        z_nbr = (mx, my, 1 - mz)

        barrier_sem = pltpu.get_barrier_semaphore()
        for nbr in (x_nbr, y_nbr, z_nbr):
            pl.semaphore_signal(barrier_sem, inc=1, device_id=nbr,
                                device_id_type=pl.DeviceIdType.MESH)
        pl.semaphore_wait(barrier_sem, 3)

        kv = k_ref[...]
        vv = v_ref[...]

        b0 = (2 * mz + my) * BL

        for k in range(BL):
            i = b0 + k
            q = q_ref[pl.ds(i, 1)][0, 0]

            page_iota = lax.broadcasted_iota(jnp.int32, (N_PAGES_SHARD, NB), 0)
            slot_iota = lax.broadcasted_iota(jnp.int32, (N_PAGES_SHARD, NB), 1)
            bt_row = bt_v_ref[pl.ds(i, 1), :]
            match = (bt_row == page_iota + mx * N_PAGES_SHARD) & (
                slot_iota < lens_ref[i]
            )
            c = jnp.sum(match.astype(jnp.float32), axis=1)
            c3 = c[:, None, None]

            s = jnp.sum(kv * q[None, None, :, :], axis=-1) * SCALE
            s_m = jnp.where(c3 > 0.0, s, NEG)

            m_loc = jnp.max(jnp.max(s_m, axis=0), axis=0)
            e = jnp.exp(s_m - m_loc[None, None, :])
            w = c3 * e
            l_loc = jnp.sum(jnp.sum(w, axis=0), axis=0)
            acc = jnp.sum(jnp.sum(w[..., None] * vv, axis=0), axis=0)

            psend[k, 0] = acc
            psend[k, 1] = jnp.broadcast_to(m_loc[:, None], (H, D))
            psend[k, 2] = jnp.broadcast_to(l_loc[:, None], (H, D))

        x_rdma = pltpu.make_async_remote_copy(
            src_ref=psend, dst_ref=precv,
            send_sem=send_sems.at[0], recv_sem=recv_sems.at[0],
            device_id=x_nbr, device_id_type=pl.DeviceIdType.MESH,
        )
        x_rdma.start()
        x_rdma.wait()

        for k in range(BL):
            acc0, m0, l0 = psend[k, 0], psend[k, 1], psend[k, 2]
            acc1, m1, l1 = precv[k, 0], precv[k, 1], precv[k, 2]
            m_new = jnp.maximum(m0, m1)
            e0 = jnp.exp(m0 - m_new)
            e1 = jnp.exp(m1 - m_new)
            merged = (acc0 * e0 + acc1 * e1) / (l0 * e0 + l1 * e1)
            out_ref[pl.ds(b0 + k, 1)] = merged[None, None]

        y_rdma = pltpu.make_async_remote_copy(
            src_ref=out_ref.at[pl.ds(b0, BL)],
            dst_ref=out_ref.at[pl.ds(b0, BL)],
            send_sem=send_sems.at[1], recv_sem=recv_sems.at[1],
            device_id=y_nbr, device_id_type=pl.DeviceIdType.MESH,
        )
        y_rdma.start()
        y_rdma.wait()

        zb0 = 4 * mz
        z_rdma = pltpu.make_async_remote_copy(
            src_ref=out_ref.at[pl.ds(zb0, 2 * BL)],
            dst_ref=out_ref.at[pl.ds(zb0, 2 * BL)],
            send_sem=send_sems.at[2], recv_sem=recv_sems.at[2],
            device_id=z_nbr, device_id_type=pl.DeviceIdType.MESH,
        )
        z_rdma.start()
        z_rdma.wait()

    out_shape = jax.ShapeDtypeStruct((B, 1, H, D), jnp.float32)
    return pl.pallas_call(
        body,
        out_shape=out_shape,
        in_specs=[
            pl.BlockSpec(memory_space=pltpu.VMEM),
            pl.BlockSpec(memory_space=pltpu.VMEM),
            pl.BlockSpec(memory_space=pltpu.VMEM),
            pl.BlockSpec(memory_space=pltpu.SMEM),
            pl.BlockSpec(memory_space=pltpu.SMEM),
            pl.BlockSpec(memory_space=pltpu.VMEM),
        ],
        out_specs=pl.BlockSpec(memory_space=pltpu.VMEM),
        scratch_shapes=[
            pltpu.VMEM((BL, 3, H, D), jnp.float32),
            pltpu.VMEM((BL, 3, H, D), jnp.float32),
            pltpu.SemaphoreType.DMA((3,)),
            pltpu.SemaphoreType.DMA((3,)),
        ],
        compiler_params=pltpu.CompilerParams(collective_id=0),
    )(Q, K, V, bt, lens, bt)


# device time: 13893 ns/iter; 12.0161x vs baseline; 1.6846x over previous
import jax
import jax.numpy as jnp
from jax import lax
from jax.experimental import pallas as pl
from jax.experimental.pallas import tpu as pltpu

P = 64
BS = 16
NK = P * BS
B = 8
H = 8
D = 64
HD = H * D
BH = B * H
NB = 64
SCALE = D ** -0.5
NEG = -1e30


def _dot(a, b):
    return jax.lax.dot_general(
        a, b, (((1,), (0,)), ((), ())), preferred_element_type=jnp.float32
    )


def kernel(Q, K, V, bt, lens):
    K2 = K.reshape(NK, HD)
    V2 = V.reshape(NK, HD)
    Qs = Q.reshape(B, H, D)
    eye = jnp.eye(H, dtype=Q.dtype)
    QM = jnp.einsum("bhd,hg->hdbg", Qs, eye).reshape(HD, BH)
    lens2 = lens.reshape(B, 1)

    def body(k_ref, v_ref, qm_ref, bt_ref, lens_ref, out_ref,
             psend, precv, send_sem, recv_sem):
        mx = lax.axis_index("x")
        my = lax.axis_index("y")
        mz = lax.axis_index("z")
        x_nbr = (1 - mx, my, mz)

        barrier_sem = pltpu.get_barrier_semaphore()
        pl.semaphore_signal(barrier_sem, inc=1, device_id=x_nbr,
                            device_id_type=pl.DeviceIdType.MESH)
        pl.semaphore_wait(barrier_sem, 1)

        e8 = (
            lax.broadcasted_iota(jnp.int32, (B, BH), 1) // H
            == lax.broadcasted_iota(jnp.int32, (B, BH), 0)
        ).astype(jnp.float32)
        g = (
            lax.broadcasted_iota(jnp.int32, (BH, HD), 0) % H
            == lax.broadcasted_iota(jnp.int32, (BH, HD), 1) // D
        ).astype(jnp.float32)
        rep = (
            lax.broadcasted_iota(jnp.int32, (NK, P), 0) // BS
            == lax.broadcasted_iota(jnp.int32, (NK, P), 1)
        ).astype(jnp.float32)

        pg = lax.broadcasted_iota(jnp.int32, (P, B, NB), 0) + mx * P
        slot = lax.broadcasted_iota(jnp.int32, (P, B, NB), 2)
        btv = bt_ref[...][None]
        lv = lens_ref[...][None]
        match = (btv == pg) & (slot < lv)
        c_pages = jnp.sum(match.astype(jnp.float32), axis=2)
        c_ab = _dot(rep, c_pages)
        c_cols = _dot(c_ab, e8)

        s_all = _dot(k_ref[...], qm_ref[...]) * SCALE
        s_m = jnp.where(c_cols > 0.0, s_all, NEG)

        m_cols = jnp.max(s_m, axis=0, keepdims=True)
        e = jnp.exp(s_m - m_cols)
        w = c_cols * e
        l_cols = jnp.sum(w, axis=0, keepdims=True)

        r = jax.lax.dot_general(
            w, v_ref[...], (((0,), (0,)), ((), ())),
            preferred_element_type=jnp.float32,
        )
        acc_flat = _dot(e8, r * g)
        m_flat = _dot(jnp.broadcast_to(m_cols, (B, BH)) * e8, g)
        l_flat = _dot(jnp.broadcast_to(l_cols, (B, BH)) * e8, g)

        psend[0] = acc_flat
        psend[1] = m_flat
        psend[2] = l_flat

        rdma = pltpu.make_async_remote_copy(
            src_ref=psend, dst_ref=precv,
            send_sem=send_sem, recv_sem=recv_sem,
            device_id=x_nbr, device_id_type=pl.DeviceIdType.MESH,
        )
        rdma.start()
        rdma.wait()

        acc0, m0, l0 = psend[0], psend[1], psend[2]
        acc1, m1, l1 = precv[0], precv[1], precv[2]
        m_new = jnp.maximum(m0, m1)
        e0 = jnp.exp(m0 - m_new)
        e1 = jnp.exp(m1 - m_new)
        out_ref[...] = (acc0 * e0 + acc1 * e1) / (l0 * e0 + l1 * e1)

    out_shape = jax.ShapeDtypeStruct((B, HD), jnp.float32)
    res = pl.pallas_call(
        body,
        out_shape=out_shape,
        in_specs=[
            pl.BlockSpec(memory_space=pltpu.VMEM),
            pl.BlockSpec(memory_space=pltpu.VMEM),
            pl.BlockSpec(memory_space=pltpu.VMEM),
            pl.BlockSpec(memory_space=pltpu.VMEM),
            pl.BlockSpec(memory_space=pltpu.VMEM),
        ],
        out_specs=pl.BlockSpec(memory_space=pltpu.VMEM),
        scratch_shapes=[
            pltpu.VMEM((3, B, HD), jnp.float32),
            pltpu.VMEM((3, B, HD), jnp.float32),
            pltpu.SemaphoreType.DMA,
            pltpu.SemaphoreType.DMA,
        ],
        compiler_params=pltpu.CompilerParams(collective_id=0),
    )(K2, V2, QM, bt, lens2)
    return res.reshape(B, 1, H, D)


# device time: 13039 ns/iter; 12.8031x vs baseline; 1.0655x over previous
import jax
import jax.numpy as jnp
from jax import lax
from jax.experimental import pallas as pl
from jax.experimental.pallas import tpu as pltpu

P = 64
BS = 16
NK = P * BS
B = 8
H = 8
D = 64
HD = H * D
BH = B * H
NB = 64
SCALE = D ** -0.5
NEG = -1e30


def _dot(a, b):
    return jax.lax.dot_general(
        a, b, (((1,), (0,)), ((), ())), preferred_element_type=jnp.float32
    )


def kernel(Q, K, V, bt, lens):
    K2 = K.astype(jnp.bfloat16).reshape(NK, HD)
    V2 = V.astype(jnp.bfloat16).reshape(NK, HD)
    Qs = Q.astype(jnp.bfloat16).reshape(B, H, D)
    eye = jnp.eye(H, dtype=jnp.bfloat16)
    QM = jnp.einsum("bhd,hg->hdbg", Qs, eye).reshape(HD, BH)
    lens2 = lens.reshape(B, 1)

    def body(k_ref, v_ref, qm_ref, bt_ref, lens_ref, out_ref,
             psend, precv, send_sem, recv_sem):
        mx = lax.axis_index("x")
        my = lax.axis_index("y")
        mz = lax.axis_index("z")
        x_nbr = (1 - mx, my, mz)

        barrier_sem = pltpu.get_barrier_semaphore()
        pl.semaphore_signal(barrier_sem, inc=1, device_id=x_nbr,
                            device_id_type=pl.DeviceIdType.MESH)

        e8 = (
            lax.broadcasted_iota(jnp.int32, (B, BH), 1) // H
            == lax.broadcasted_iota(jnp.int32, (B, BH), 0)
        ).astype(jnp.float32)
        g = (
            lax.broadcasted_iota(jnp.int32, (BH, HD), 0) % H
            == lax.broadcasted_iota(jnp.int32, (BH, HD), 1) // D
        ).astype(jnp.float32)
        rep = (
            lax.broadcasted_iota(jnp.int32, (NK, P), 0) // BS
            == lax.broadcasted_iota(jnp.int32, (NK, P), 1)
        ).astype(jnp.float32)

        pg = lax.broadcasted_iota(jnp.int32, (P, B, NB), 0) + mx * P
        slot = lax.broadcasted_iota(jnp.int32, (P, B, NB), 2)
        btv = bt_ref[...][None]
        lv = lens_ref[...][None]
        match = (btv == pg) & (slot < lv)
        c_pages = jnp.sum(match.astype(jnp.float32), axis=2)
        c_ab = _dot(rep, c_pages)
        c_cols = _dot(c_ab, e8)

        s_all = _dot(k_ref[...], qm_ref[...]) * SCALE
        s_m = jnp.where(c_cols > 0.0, s_all, NEG)

        m_cols = jnp.max(s_m, axis=0, keepdims=True)
        e = jnp.exp(s_m - m_cols)
        w = c_cols * e
        l_cols = jnp.sum(w, axis=0, keepdims=True)

        r = jax.lax.dot_general(
            w.astype(jnp.bfloat16), v_ref[...], (((0,), (0,)), ((), ())),
            preferred_element_type=jnp.float32,
        )
        acc_flat = _dot(e8, r * g)
        m_flat = _dot(jnp.broadcast_to(m_cols, (B, BH)) * e8, g)
        l_flat = _dot(jnp.broadcast_to(l_cols, (B, BH)) * e8, g)

        psend[0] = acc_flat
        psend[1] = m_flat
        psend[2] = l_flat

        pl.semaphore_wait(barrier_sem, 1)
        rdma = pltpu.make_async_remote_copy(
            src_ref=psend, dst_ref=precv,
            send_sem=send_sem, recv_sem=recv_sem,
            device_id=x_nbr, device_id_type=pl.DeviceIdType.MESH,
        )
        rdma.start()
        rdma.wait()

        acc0, m0, l0 = psend[0], psend[1], psend[2]
        acc1, m1, l1 = precv[0], precv[1], precv[2]
        m_new = jnp.maximum(m0, m1)
        e0 = jnp.exp(m0 - m_new)
        e1 = jnp.exp(m1 - m_new)
        out_ref[...] = (acc0 * e0 + acc1 * e1) / (l0 * e0 + l1 * e1)

    out_shape = jax.ShapeDtypeStruct((B, HD), jnp.float32)
    res = pl.pallas_call(
        body,
        out_shape=out_shape,
        in_specs=[
            pl.BlockSpec(memory_space=pltpu.VMEM),
            pl.BlockSpec(memory_space=pltpu.VMEM),
            pl.BlockSpec(memory_space=pltpu.VMEM),
            pl.BlockSpec(memory_space=pltpu.VMEM),
            pl.BlockSpec(memory_space=pltpu.VMEM),
        ],
        out_specs=pl.BlockSpec(memory_space=pltpu.VMEM),
        scratch_shapes=[
            pltpu.VMEM((3, B, HD), jnp.float32),
            pltpu.VMEM((3, B, HD), jnp.float32),
            pltpu.SemaphoreType.DMA,
            pltpu.SemaphoreType.DMA,
        ],
        compiler_params=pltpu.CompilerParams(collective_id=0),
    )(K2, V2, QM, bt, lens2)
    return res.reshape(B, 1, H, D)
